# baseline (speedup 1.0000x reference)
import jax
import jax.numpy as jnp
from jax import lax
from jax.experimental import pallas as pl
from jax.experimental.pallas import tpu as pltpu
from jax.experimental.pallas import tpu_sc as plsc

N_NODES = 10000
N_EDGES = 320000
D_FEAT = 128
D_EDGE = 16
D_OUT = 128

NC = 2
NS = 16
EPAD = 327680
EPC = EPAD // NC
EPT = EPC // NS
K = 128
NCH = EPT // K
G = 16
NGRP = NCH // G
GK = G * K
RPT = 640
NROW = NS * RPT
DA = D_FEAT
TRASH = N_NODES + 16

BM = 1000


def _zero_fill(ref, rows):
    z = jnp.zeros((16,), jnp.float32)

    def body(i, carry):
        for j in range(DA // 16):
            ref[i, pl.ds(j * 16, 16)] = z
        return carry

    lax.fori_loop(0, rows, body, 0)


def _sc_segsum(src_h, dst2_h, x_h, efp_h, outa_h, outb_h,
               acc, sidxg, didxg, rows, packed, sg0, sg1, ss0, ss1):
    cid = lax.axis_index("c")
    sid = lax.axis_index("s")
    sem_g = (sg0, sg1)
    sem_s = (ss0, ss1)

    def zero_acc():
        _zero_fill(rows.at[0], K)
        for t in range(RPT // K):
            pltpu.sync_copy(rows.at[0], acc.at[pl.ds(sid * RPT + t * K, K)])

    def drain(out_h):
        base = sid * RPT
        pltpu.sync_copy(acc.at[pl.ds(base, RPT)],
                        out_h.at[pl.ds(cid * NROW + base, RPT)])

    zero_acc()
    plsc.subcore_barrier()

    def group_a(gg, carry):
        geb = pl.multiple_of(cid * EPC + sid * EPT + gg * GK, GK)
        grow = pl.multiple_of(geb // K, 8)
        pltpu.sync_copy(src_h.at[pl.ds(geb, GK)], sidxg)
        pltpu.sync_copy(dst2_h.at[pl.ds(grow, G)], didxg)
        cp_g = [None, None]
        cp_s = [None, None]
        cp_g[0] = pltpu.async_copy(
            x_h.at[sidxg.at[pl.ds(0, K)]], rows.at[0], sem_g[0])
        cp_g[1] = pltpu.async_copy(
            x_h.at[sidxg.at[pl.ds(K, K)]], rows.at[1], sem_g[1])
        for b in range(G):
            cur = b % 2
            cp_g[cur].wait()
            cp_s[cur] = pltpu.async_copy(
                rows.at[cur], acc.at[didxg.at[b]], sem_s[cur], add=True)
            if b + 2 < G:
                cp_s[cur].wait()
                cp_g[cur] = pltpu.async_copy(
                    x_h.at[sidxg.at[pl.ds((b + 2) * K, K)]],
                    rows.at[cur], sem_g[cur])
        cp_s[0].wait()
        cp_s[1].wait()
        return carry

    lax.fori_loop(0, NGRP, group_a, 0)
    plsc.subcore_barrier()
    drain(outa_h)
    plsc.subcore_barrier()

    zero_acc()
    one0 = jnp.where(jnp.arange(16, dtype=jnp.int32) == 0,
                     jnp.float32(1.0), jnp.float32(0.0))
    _zero_fill(rows.at[0], K)
    _zero_fill(rows.at[1], K)

    def preset(i, carry):
        rows[0, i, pl.ds(D_EDGE, 16)] = one0
        rows[1, i, pl.ds(D_EDGE, 16)] = one0
        return carry

    lax.fori_loop(0, K, preset, 0)
    plsc.subcore_barrier()

    def group_b(gg, carry):
        geb = pl.multiple_of(cid * EPC + sid * EPT + gg * GK, GK)
        grow = pl.multiple_of(geb // K, 8)
        pltpu.sync_copy(dst2_h.at[pl.ds(grow, G)], didxg)
        cp_s = [None, None]
        for b in range(G):
            cur = b % 2
            prow = pl.multiple_of(geb // 4 + b * (K // 4), K // 4)
            pltpu.sync_copy(efp_h.at[pl.ds(prow, K // 4)], packed)
            if b >= 2:
                cp_s[cur].wait()
            buf = rows.at[cur]

            def repack(i, c):
                buf[i, pl.ds(0, 16)] = packed[i // 4, pl.ds((i % 4) * 32, 16)]
                return c

            lax.fori_loop(0, K, repack, 0)
            cp_s[cur] = pltpu.async_copy(
                buf, acc.at[didxg.at[b]], sem_s[cur], add=True)
        cp_s[0].wait()
        cp_s[1].wait()
        return carry

    lax.fori_loop(0, NGRP, group_b, 0)
    plsc.subcore_barrier()
    drain(outb_h)


_sc_call = pl.kernel(
    _sc_segsum,
    out_type=(jax.ShapeDtypeStruct((NC * NROW, DA), jnp.float32),
              jax.ShapeDtypeStruct((NC * NROW, DA), jnp.float32)),
    mesh=plsc.VectorSubcoreMesh(core_axis_name="c", subcore_axis_name="s",
                                num_cores=NC, num_subcores=NS),
    scratch_types=[
        pltpu.VMEM_SHARED((NROW, DA), jnp.float32),
        pltpu.VMEM((GK,), jnp.int32),
        pltpu.VMEM((G, K), jnp.int32),
        pltpu.VMEM((2, K, DA), jnp.float32),
        pltpu.VMEM((K // 4, DA), jnp.float32),
        pltpu.SemaphoreType.DMA,
        pltpu.SemaphoreType.DMA,
        pltpu.SemaphoreType.DMA,
        pltpu.SemaphoreType.DMA,
    ],
)


def _tc_finish(a_ref, bb_ref, w_ref, bias_ref, o_ref):
    a = a_ref[0] + a_ref[1]
    bb = bb_ref[0] + bb_ref[1]
    cnt = bb[:, D_EDGE:D_EDGE + 1]
    h = jnp.dot(a, w_ref[:D_FEAT, :], preferred_element_type=jnp.float32)
    h = h + jnp.dot(bb[:, :D_EDGE], w_ref[D_FEAT:, :],
                    preferred_element_type=jnp.float32)
    h = h + cnt * bias_ref[...]
    o_ref[...] = h / jnp.maximum(cnt, 1.0)


_tc_call = pl.pallas_call(
    _tc_finish,
    grid=(N_NODES // BM,),
    in_specs=[
        pl.BlockSpec((NC, BM, DA), lambda i: (0, i, 0)),
        pl.BlockSpec((NC, BM, DA), lambda i: (0, i, 0)),
        pl.BlockSpec((D_FEAT + D_EDGE, D_OUT), lambda i: (0, 0)),
        pl.BlockSpec((1, D_OUT), lambda i: (0, 0)),
    ],
    out_specs=pl.BlockSpec((BM, D_OUT), lambda i: (i, 0)),
    out_shape=jax.ShapeDtypeStruct((N_NODES, D_OUT), jnp.float32),
)


def kernel(x, edge_index, edge_features, W, b):
    npad = EPAD - N_EDGES
    src = jnp.concatenate([edge_index[0].astype(jnp.int32),
                           jnp.zeros((npad,), jnp.int32)])
    dst = jnp.concatenate([edge_index[1].astype(jnp.int32),
                           jnp.full((npad,), TRASH, jnp.int32)])
    dst2 = dst.reshape(EPAD // K, K)
    efa = jnp.concatenate([edge_features.astype(jnp.float32),
                           jnp.ones((N_EDGES, 1), jnp.float32),
                           jnp.zeros((N_EDGES, 15), jnp.float32)], axis=1)
    efa = jnp.concatenate([efa, jnp.zeros((npad, 32), jnp.float32)], axis=0)
    efp = efa.reshape(EPAD // 4, 128)
    pa, pb = _sc_call(src, dst2, x, efp)
    pa = pa.reshape(NC, NROW, DA)
    pb = pb.reshape(NC, NROW, DA)
    return _tc_call(pa, pb, W, b.reshape(1, D_OUT))


# speedup vs baseline: 1.1745x; 1.1745x over previous
import jax
import jax.numpy as jnp
from jax import lax
from jax.experimental import pallas as pl
from jax.experimental.pallas import tpu as pltpu
from jax.experimental.pallas import tpu_sc as plsc

N_NODES = 10000
N_EDGES = 320000
D_FEAT = 128
D_EDGE = 16
D_OUT = 128

NC = 2
NS = 16
EPAD = 327680
EPT = EPAD // NS
K = 128
NCH = EPT // K
G = 16
NGRP = NCH // G
GK = G * K
RPT = 640
NROW = NS * RPT
DA = D_FEAT
TRASH = N_NODES + 16

BM = 1000


def _zero_fill(ref, rows):
    z = jnp.zeros((16,), jnp.float32)

    def body(i, carry):
        for j in range(DA // 16):
            ref[i, pl.ds(j * 16, 16)] = z
        return carry

    lax.fori_loop(0, rows, body, 0)


def _sc_segsum(src_h, dst2_h, x_h, efp_h, outa_h, outb_h,
               acc, sidxg, didxg, rows, packed, sg0, sg1, ss0, ss1):
    cid = lax.axis_index("c")
    sid = lax.axis_index("s")
    sem_g = (sg0, sg1)
    sem_s = (ss0, ss1)

    _zero_fill(rows.at[0], K)
    for t in range(RPT // K):
        pltpu.sync_copy(rows.at[0], acc.at[pl.ds(sid * RPT + t * K, K)])

    @pl.when(cid == 1)
    def _preset():
        one0 = jnp.where(jnp.arange(16, dtype=jnp.int32) == 0,
                         jnp.float32(1.0), jnp.float32(0.0))
        _zero_fill(rows.at[1], K)

        def body(i, carry):
            rows[0, i, pl.ds(D_EDGE, 16)] = one0
            rows[1, i, pl.ds(D_EDGE, 16)] = one0
            return carry

        lax.fori_loop(0, K, body, 0)

    plsc.subcore_barrier()

    @pl.when(cid == 0)
    def _edges_a():
        def group(gg, carry):
            geb = pl.multiple_of(sid * EPT + gg * GK, GK)
            grow = pl.multiple_of(sid * NCH + gg * G, 8)
            pltpu.sync_copy(src_h.at[pl.ds(geb, GK)], sidxg)
            pltpu.sync_copy(dst2_h.at[pl.ds(grow, G)], didxg)
            cp_g = [None, None]
            cp_s = [None, None]
            cp_g[0] = pltpu.async_copy(
                x_h.at[sidxg.at[pl.ds(0, K)]], rows.at[0], sem_g[0])
            cp_g[1] = pltpu.async_copy(
                x_h.at[sidxg.at[pl.ds(K, K)]], rows.at[1], sem_g[1])
            for b in range(G):
                cur = b % 2
                cp_g[cur].wait()
                cp_s[cur] = pltpu.async_copy(
                    rows.at[cur], acc.at[didxg.at[b]], sem_s[cur], add=True)
                if b + 2 < G:
                    cp_s[cur].wait()
                    cp_g[cur] = pltpu.async_copy(
                        x_h.at[sidxg.at[pl.ds((b + 2) * K, K)]],
                        rows.at[cur], sem_g[cur])
            cp_s[0].wait()
            cp_s[1].wait()
            return carry

        lax.fori_loop(0, NGRP, group, 0)

    @pl.when(cid == 1)
    def _edges_b():
        def group(gg, carry):
            grow = pl.multiple_of(sid * NCH + gg * G, 8)
            pltpu.sync_copy(dst2_h.at[pl.ds(grow, G)], didxg)
            cp_s = [None, None]
            for b in range(G):
                cur = b % 2
                prow = pl.multiple_of(
                    jnp.minimum((sid * EPT + (gg * G + b) * K) // 8,
                                N_EDGES // 8 - K // 8), K // 8)
                pltpu.sync_copy(efp_h.at[pl.ds(prow, K // 8)], packed)
                if b >= 2:
                    cp_s[cur].wait()
                buf = rows.at[cur]

                def repack(i, c):
                    buf[i, pl.ds(0, 16)] = packed[i // 8, pl.ds((i % 8) * 16, 16)]
                    return c

                lax.fori_loop(0, K, repack, 0)
                cp_s[cur] = pltpu.async_copy(
                    buf, acc.at[didxg.at[b]], sem_s[cur], add=True)
            cp_s[0].wait()
            cp_s[1].wait()
            return carry

        lax.fori_loop(0, NGRP, group, 0)

    plsc.subcore_barrier()

    base = sid * RPT

    @pl.when(cid == 0)
    def _drain_a():
        pltpu.sync_copy(acc.at[pl.ds(base, RPT)], outa_h.at[pl.ds(base, RPT)])

    @pl.when(cid == 1)
    def _drain_b():
        pltpu.sync_copy(acc.at[pl.ds(base, RPT)], outb_h.at[pl.ds(base, RPT)])


_sc_call = pl.kernel(
    _sc_segsum,
    out_type=(jax.ShapeDtypeStruct((NROW, DA), jnp.float32),
              jax.ShapeDtypeStruct((NROW, DA), jnp.float32)),
    mesh=plsc.VectorSubcoreMesh(core_axis_name="c", subcore_axis_name="s",
                                num_cores=NC, num_subcores=NS),
    scratch_types=[
        pltpu.VMEM_SHARED((NROW, DA), jnp.float32),
        pltpu.VMEM((GK,), jnp.int32),
        pltpu.VMEM((G, K), jnp.int32),
        pltpu.VMEM((2, K, DA), jnp.float32),
        pltpu.VMEM((K // 8, DA), jnp.float32),
        pltpu.SemaphoreType.DMA,
        pltpu.SemaphoreType.DMA,
        pltpu.SemaphoreType.DMA,
        pltpu.SemaphoreType.DMA,
    ],
)


def _tc_finish(a_ref, bb_ref, w_ref, bias_ref, o_ref):
    a = a_ref[...]
    bb = bb_ref[...]
    cnt = bb[:, D_EDGE:D_EDGE + 1]
    h = jnp.dot(a, w_ref[:D_FEAT, :], preferred_element_type=jnp.float32)
    h = h + jnp.dot(bb[:, :D_EDGE], w_ref[D_FEAT:, :],
                    preferred_element_type=jnp.float32)
    h = h + cnt * bias_ref[...]
    o_ref[...] = h / jnp.maximum(cnt, 1.0)


_tc_call = pl.pallas_call(
    _tc_finish,
    grid=(N_NODES // BM,),
    in_specs=[
        pl.BlockSpec((BM, DA), lambda i: (i, 0)),
        pl.BlockSpec((BM, DA), lambda i: (i, 0)),
        pl.BlockSpec((D_FEAT + D_EDGE, D_OUT), lambda i: (0, 0)),
        pl.BlockSpec((1, D_OUT), lambda i: (0, 0)),
    ],
    out_specs=pl.BlockSpec((BM, D_OUT), lambda i: (i, 0)),
    out_shape=jax.ShapeDtypeStruct((N_NODES, D_OUT), jnp.float32),
)


def kernel(x, edge_index, edge_features, W, b):
    npad = EPAD - N_EDGES
    src = jnp.concatenate([edge_index[0].astype(jnp.int32),
                           jnp.zeros((npad,), jnp.int32)])
    dst = jnp.concatenate([edge_index[1].astype(jnp.int32),
                           jnp.full((npad,), TRASH, jnp.int32)])
    dst2 = dst.reshape(EPAD // K, K)
    efp = edge_features.astype(jnp.float32).reshape(N_EDGES // 8, 128)
    pa, pb = _sc_call(src, dst2, x, efp)
    return _tc_call(pa, pb, W, b.reshape(1, D_OUT))


# speedup vs baseline: 1.3867x; 1.1807x over previous
import jax
import jax.numpy as jnp
from jax import lax
from jax.experimental import pallas as pl
from jax.experimental.pallas import tpu as pltpu
from jax.experimental.pallas import tpu_sc as plsc

N_NODES = 10000
N_EDGES = 320000
D_FEAT = 128
D_EDGE = 16
D_OUT = 128

NC = 2
NS = 16
EPAD = 327680
EPT = EPAD // NS
K = 128
KH = K // 2
XW = 64
NCH = EPT // K
G = 8
NGRP = NCH // G
GK = G * K
RPT = 640
NROW = NS * RPT
DA = D_FEAT
TRASH = N_NODES + 16

BM = 1000


def _zero_fill(ref, rows):
    z = jnp.zeros((16,), jnp.float32)

    def body(i, carry):
        for j in range(DA // 16):
            ref[i, pl.ds(j * 16, 16)] = z
        return carry

    lax.fori_loop(0, rows, body, 0)


def _sc_segsum(src_h, dst2_h, xp_h, efp_h, outa_h, outb_h,
               acc, sidxg, didxg, gbuf, fbuf, packed, sg0, sg1, ss0, ss1):
    cid = lax.axis_index("c")
    sid = lax.axis_index("s")
    sem_g = (sg0, sg1)
    sem_s = (ss0, ss1)

    _zero_fill(fbuf.at[0], K)
    for t in range(RPT // K):
        pltpu.sync_copy(fbuf.at[0], acc.at[pl.ds(sid * RPT + t * K, K)])

    @pl.when(cid == 1)
    def _preset():
        one0 = jnp.where(jnp.arange(16, dtype=jnp.int32) == 0,
                         jnp.float32(1.0), jnp.float32(0.0))
        _zero_fill(fbuf.at[1], K)

        def body(i, carry):
            fbuf[0, i, pl.ds(D_EDGE, 16)] = one0
            fbuf[1, i, pl.ds(D_EDGE, 16)] = one0
            return carry

        lax.fori_loop(0, K, body, 0)

    plsc.subcore_barrier()

    @pl.when(cid == 0)
    def _edges_a():
        def unpack_half(h, fb, base):
            gb = gbuf.at[h]

            def body(i, c):
                for j in range(XW // 16):
                    w = gb[i, pl.ds(j * 16, 16)]
                    v = plsc.bitcast(w, jnp.bfloat16)
                    lo, hi = plsc.unpack(
                        v, format=plsc.PackFormat.INTERLEAVED,
                        preferred_element_type=jnp.float32)
                    fb[base + i, pl.ds(j * 32, 16)] = lo
                    fb[base + i, pl.ds(j * 32 + 16, 16)] = hi
                return c

            lax.fori_loop(0, KH, body, 0)

        def glaunch(b, h):
            return pltpu.async_copy(
                xp_h.at[sidxg.at[pl.ds(b * K + h * KH, KH)]],
                gbuf.at[h], sem_g[h])

        def group(gg, carry):
            geb = pl.multiple_of(sid * EPT + gg * GK, GK)
            grow = pl.multiple_of(sid * NCH + gg * G, 8)
            pltpu.sync_copy(src_h.at[pl.ds(geb, GK)], sidxg)
            pltpu.sync_copy(dst2_h.at[pl.ds(grow, G)], didxg)
            cp_g = [glaunch(0, 0), glaunch(0, 1)]
            cp_s = [None, None]
            for b in range(G):
                cur = b % 2
                if b >= 2:
                    cp_s[cur].wait()
                fb = fbuf.at[cur]
                cp_g[0].wait()
                unpack_half(0, fb, 0)
                if b + 1 < G:
                    cp_g[0] = glaunch(b + 1, 0)
                cp_g[1].wait()
                unpack_half(1, fb, KH)
                if b + 1 < G:
                    cp_g[1] = glaunch(b + 1, 1)
                cp_s[cur] = pltpu.async_copy(
                    fb, acc.at[didxg.at[b]], sem_s[cur], add=True)
            cp_s[0].wait()
            cp_s[1].wait()
            return carry

        lax.fori_loop(0, NGRP, group, 0)

    @pl.when(cid == 1)
    def _edges_b():
        def group(gg, carry):
            grow = pl.multiple_of(sid * NCH + gg * G, 8)
            pltpu.sync_copy(dst2_h.at[pl.ds(grow, G)], didxg)
            cp_s = [None, None]
            for b in range(G):
                cur = b % 2
                prow = pl.multiple_of(
                    jnp.minimum((sid * EPT + (gg * G + b) * K) // 8,
                                N_EDGES // 8 - K // 8), K // 8)
                pltpu.sync_copy(efp_h.at[pl.ds(prow, K // 8)], packed)
                if b >= 2:
                    cp_s[cur].wait()
                buf = fbuf.at[cur]

                def repack(i, c):
                    buf[i, pl.ds(0, 16)] = packed[i // 8, pl.ds((i % 8) * 16, 16)]
                    return c

                lax.fori_loop(0, K, repack, 0)
                cp_s[cur] = pltpu.async_copy(
                    buf, acc.at[didxg.at[b]], sem_s[cur], add=True)
            cp_s[0].wait()
            cp_s[1].wait()
            return carry

        lax.fori_loop(0, NGRP, group, 0)

    plsc.subcore_barrier()

    base = sid * RPT

    @pl.when(cid == 0)
    def _drain_a():
        pltpu.sync_copy(acc.at[pl.ds(base, RPT)], outa_h.at[pl.ds(base, RPT)])

    @pl.when(cid == 1)
    def _drain_b():
        pltpu.sync_copy(acc.at[pl.ds(base, RPT)], outb_h.at[pl.ds(base, RPT)])


_sc_call = pl.kernel(
    _sc_segsum,
    out_type=(jax.ShapeDtypeStruct((NROW, DA), jnp.float32),
              jax.ShapeDtypeStruct((NROW, DA), jnp.float32)),
    mesh=plsc.VectorSubcoreMesh(core_axis_name="c", subcore_axis_name="s",
                                num_cores=NC, num_subcores=NS),
    compiler_params=pltpu.CompilerParams(use_tc_tiling_on_sc=False,
                                         needs_layout_passes=False),
    scratch_types=[
        pltpu.VMEM_SHARED((NROW, DA), jnp.float32),
        pltpu.VMEM((GK,), jnp.int32),
        pltpu.VMEM((G, K), jnp.int32),
        pltpu.VMEM((2, KH, XW), jnp.int32),
        pltpu.VMEM((2, K, DA), jnp.float32),
        pltpu.VMEM((K // 8, DA), jnp.float32),
        pltpu.SemaphoreType.DMA,
        pltpu.SemaphoreType.DMA,
        pltpu.SemaphoreType.DMA,
        pltpu.SemaphoreType.DMA,
    ],
)


def _tc_finish(a_ref, bb_ref, w_ref, bias_ref, o_ref):
    a = a_ref[...]
    bb = bb_ref[...]
    cnt = bb[:, D_EDGE:D_EDGE + 1]
    h = jnp.dot(a, w_ref[:D_FEAT, :], preferred_element_type=jnp.float32)
    h = h + jnp.dot(bb[:, :D_EDGE], w_ref[D_FEAT:, :],
                    preferred_element_type=jnp.float32)
    h = h + cnt * bias_ref[...]
    o_ref[...] = h / jnp.maximum(cnt, 1.0)


_tc_call = pl.pallas_call(
    _tc_finish,
    grid=(N_NODES // BM,),
    in_specs=[
        pl.BlockSpec((BM, DA), lambda i: (i, 0)),
        pl.BlockSpec((BM, DA), lambda i: (i, 0)),
        pl.BlockSpec((D_FEAT + D_EDGE, D_OUT), lambda i: (0, 0)),
        pl.BlockSpec((1, D_OUT), lambda i: (0, 0)),
    ],
    out_specs=pl.BlockSpec((BM, D_OUT), lambda i: (i, 0)),
    out_shape=jax.ShapeDtypeStruct((N_NODES, D_OUT), jnp.float32),
)


def kernel(x, edge_index, edge_features, W, b):
    npad = EPAD - N_EDGES
    src = jnp.concatenate([edge_index[0].astype(jnp.int32),
                           jnp.zeros((npad,), jnp.int32)])
    dst = jnp.concatenate([edge_index[1].astype(jnp.int32),
                           jnp.full((npad,), TRASH, jnp.int32)])
    dst2 = dst.reshape(EPAD // K, K)
    xb = x.astype(jnp.bfloat16).reshape(N_NODES, 4, 2, 16)
    xp = jax.lax.bitcast_convert_type(xb.transpose(0, 1, 3, 2),
                                      jnp.int32).reshape(N_NODES, XW)
    efp = edge_features.astype(jnp.float32).reshape(N_EDGES // 8, 128)
    pa, pb = _sc_call(src, dst2, xp, efp)
    return _tc_call(pa, pb, W, b.reshape(1, D_OUT))


# speedup vs baseline: 1.3868x; 1.0001x over previous
import jax
import jax.numpy as jnp
from jax import lax
from jax.experimental import pallas as pl
from jax.experimental.pallas import tpu as pltpu
from jax.experimental.pallas import tpu_sc as plsc

N_NODES = 10000
N_EDGES = 320000
D_FEAT = 128
D_EDGE = 16
D_OUT = 128

NC = 2
NS = 16
EPAD = 327680
EPT = EPAD // NS
K = 128
KH = K // 2
XW = 64
NCH = EPT // K
G = 8
NGRP = NCH // G
GK = G * K
RPT = 640
NROW = NS * RPT
DA = D_FEAT
TRASH = N_NODES + 16

BM = 1000


def _zero_fill(ref, rows):
    z = jnp.zeros((16,), jnp.float32)

    def body(i, carry):
        for j in range(DA // 16):
            ref[i, pl.ds(j * 16, 16)] = z
        return carry

    lax.fori_loop(0, rows, body, 0)


def _sc_segsum(src_h, dst2_h, xp_h, efp_h, outa_h, outb_h,
               acc, sidxg, didxg, gbuf, fbuf, packed,
               sg0, sg1, sg2, ss0, ss1):
    cid = lax.axis_index("c")
    sid = lax.axis_index("s")
    sem_g = (sg0, sg1, sg2)
    sem_s = (ss0, ss1)

    _zero_fill(fbuf.at[0], K)
    for t in range(RPT // K):
        pltpu.sync_copy(fbuf.at[0], acc.at[pl.ds(sid * RPT + t * K, K)])

    @pl.when(cid == 1)
    def _preset():
        one0 = jnp.where(jnp.arange(16, dtype=jnp.int32) == 0,
                         jnp.float32(1.0), jnp.float32(0.0))
        _zero_fill(fbuf.at[1], K)

        def body(i, carry):
            fbuf[0, i, pl.ds(D_EDGE, 16)] = one0
            fbuf[1, i, pl.ds(D_EDGE, 16)] = one0
            return carry

        lax.fori_loop(0, K, body, 0)

    plsc.subcore_barrier()

    @pl.when(cid == 0)
    def _edges_a():
        def unpack_half(h, fb, base):
            gb = gbuf.at[h]

            def body(i, c):
                for j in range(XW // 16):
                    w = gb[i, pl.ds(j * 16, 16)]
                    v = plsc.bitcast(w, jnp.bfloat16)
                    lo, hi = plsc.unpack(
                        v, format=plsc.PackFormat.INTERLEAVED,
                        preferred_element_type=jnp.float32)
                    fb[base + i, pl.ds(j * 32, 16)] = lo
                    fb[base + i, pl.ds(j * 32 + 16, 16)] = hi
                return c

            lax.fori_loop(0, KH, body, 0)

        def glaunch(b, h):
            slot = (2 * b + h) % 3
            return pltpu.async_copy(
                xp_h.at[sidxg.at[pl.ds(b * K + h * KH, KH)]],
                gbuf.at[slot], sem_g[slot])

        def group(gg, carry):
            geb = pl.multiple_of(sid * EPT + gg * GK, GK)
            grow = pl.multiple_of(sid * NCH + gg * G, 8)
            pltpu.sync_copy(src_h.at[pl.ds(geb, GK)], sidxg)
            pltpu.sync_copy(dst2_h.at[pl.ds(grow, G)], didxg)
            cp_g = [glaunch(0, 0), glaunch(0, 1), glaunch(1, 0)]
            cp_s = [None, None]
            for b in range(G):
                cur = b % 2
                s0 = (2 * b) % 3
                s1 = (2 * b + 1) % 3
                if b >= 2:
                    cp_s[cur].wait()
                fb = fbuf.at[cur]
                cp_g[s0].wait()
                unpack_half(s0, fb, 0)
                if 2 * b + 3 < 2 * G:
                    cp_g[s0] = glaunch((b * 2 + 3) // 2, (b * 2 + 3) % 2)
                cp_g[s1].wait()
                unpack_half(s1, fb, KH)
                if 2 * b + 4 < 2 * G:
                    cp_g[s1] = glaunch((b * 2 + 4) // 2, (b * 2 + 4) % 2)
                cp_s[cur] = pltpu.async_copy(
                    fb, acc.at[didxg.at[b]], sem_s[cur], add=True)
            cp_s[0].wait()
            cp_s[1].wait()
            return carry

        lax.fori_loop(0, NGRP, group, 0)

    @pl.when(cid == 1)
    def _edges_b():
        def group(gg, carry):
            grow = pl.multiple_of(sid * NCH + gg * G, 8)
            pltpu.sync_copy(dst2_h.at[pl.ds(grow, G)], didxg)
            cp_s = [None, None]
            for b in range(G):
                cur = b % 2
                prow = pl.multiple_of(
                    jnp.minimum(sid * EPT + (gg * G + b) * K,
                                N_EDGES - K), K)
                pltpu.sync_copy(efp_h.at[pl.ds(prow, K)], packed)
                if b >= 2:
                    cp_s[cur].wait()
                buf = fbuf.at[cur]

                def repack(i, c):
                    buf[i, pl.ds(0, 16)] = packed[i, pl.ds(0, 16)]
                    return c

                lax.fori_loop(0, K, repack, 0)
                cp_s[cur] = pltpu.async_copy(
                    buf, acc.at[didxg.at[b]], sem_s[cur], add=True)
            cp_s[0].wait()
            cp_s[1].wait()
            return carry

        lax.fori_loop(0, NGRP, group, 0)

    plsc.subcore_barrier()

    base = sid * RPT

    @pl.when(cid == 0)
    def _drain_a():
        pltpu.sync_copy(acc.at[pl.ds(base, RPT)], outa_h.at[pl.ds(base, RPT)])

    @pl.when(cid == 1)
    def _drain_b():
        pltpu.sync_copy(acc.at[pl.ds(base, RPT)], outb_h.at[pl.ds(base, RPT)])


_sc_call = pl.kernel(
    _sc_segsum,
    out_type=(jax.ShapeDtypeStruct((NROW, DA), jnp.float32),
              jax.ShapeDtypeStruct((NROW, DA), jnp.float32)),
    mesh=plsc.VectorSubcoreMesh(core_axis_name="c", subcore_axis_name="s",
                                num_cores=NC, num_subcores=NS),
    compiler_params=pltpu.CompilerParams(use_tc_tiling_on_sc=False,
                                         needs_layout_passes=False),
    scratch_types=[
        pltpu.VMEM_SHARED((NROW, DA), jnp.float32),
        pltpu.VMEM((GK,), jnp.int32),
        pltpu.VMEM((G, K), jnp.int32),
        pltpu.VMEM((3, KH, XW), jnp.int32),
        pltpu.VMEM((2, K, DA), jnp.float32),
        pltpu.VMEM((K, D_EDGE), jnp.float32),
        pltpu.SemaphoreType.DMA,
        pltpu.SemaphoreType.DMA,
        pltpu.SemaphoreType.DMA,
        pltpu.SemaphoreType.DMA,
        pltpu.SemaphoreType.DMA,
    ],
)


def _tc_finish(a_ref, bb_ref, w_ref, bias_ref, o_ref):
    a = a_ref[...]
    bb = bb_ref[...]
    cnt = bb[:, D_EDGE:D_EDGE + 1]
    h = jnp.dot(a, w_ref[:D_FEAT, :], preferred_element_type=jnp.float32)
    h = h + jnp.dot(bb[:, :D_EDGE], w_ref[D_FEAT:, :],
                    preferred_element_type=jnp.float32)
    h = h + cnt * bias_ref[...]
    o_ref[...] = h / jnp.maximum(cnt, 1.0)


_tc_call = pl.pallas_call(
    _tc_finish,
    grid=(N_NODES // BM,),
    in_specs=[
        pl.BlockSpec((BM, DA), lambda i: (i, 0)),
        pl.BlockSpec((BM, DA), lambda i: (i, 0)),
        pl.BlockSpec((D_FEAT + D_EDGE, D_OUT), lambda i: (0, 0)),
        pl.BlockSpec((1, D_OUT), lambda i: (0, 0)),
    ],
    out_specs=pl.BlockSpec((BM, D_OUT), lambda i: (i, 0)),
    out_shape=jax.ShapeDtypeStruct((N_NODES, D_OUT), jnp.float32),
)


def kernel(x, edge_index, edge_features, W, b):
    npad = EPAD - N_EDGES
    src = jnp.concatenate([edge_index[0].astype(jnp.int32),
                           jnp.zeros((npad,), jnp.int32)])
    dst = jnp.concatenate([edge_index[1].astype(jnp.int32),
                           jnp.full((npad,), TRASH, jnp.int32)])
    dst2 = dst.reshape(EPAD // K, K)
    xb = x.astype(jnp.bfloat16).reshape(N_NODES, 4, 2, 16)
    xp = jax.lax.bitcast_convert_type(xb.transpose(0, 1, 3, 2),
                                      jnp.int32).reshape(N_NODES, XW)
    efp = edge_features.astype(jnp.float32)
    pa, pb = _sc_call(src, dst2, xp, efp)
    return _tc_call(pa, pb, W, b.reshape(1, D_OUT))


# speedup vs baseline: 1.4191x; 1.0233x over previous
import jax
import jax.numpy as jnp
from jax import lax
from jax.experimental import pallas as pl
from jax.experimental.pallas import tpu as pltpu
from jax.experimental.pallas import tpu_sc as plsc

N_NODES = 10000
N_EDGES = 320000
D_FEAT = 128
D_EDGE = 16
D_OUT = 128

NC = 2
NS = 16
EPAD = 327680
EPT = EPAD // NS
K = 128
KH = K // 2
XW = 64
NCH = EPT // K
G = 8
NGRP = NCH // G
GK = G * K
RPT = 640
NROW = NS * RPT
DA = D_FEAT
TRASH = N_NODES + 16

BM = 1000


def _zero_fill(ref, rows):
    z = jnp.zeros((16,), jnp.float32)

    def body(i, carry):
        for j in range(DA // 16):
            ref[i, pl.ds(j * 16, 16)] = z
        return carry

    lax.fori_loop(0, rows, body, 0)


def _sc_segsum(src_h, dst2_h, xp_h, efp_h, outa_h, outb_h,
               acc, sidxg, didxg, gbuf, fbuf, packed, sg0, sg1, ss0, ss1):
    cid = lax.axis_index("c")
    sid = lax.axis_index("s")
    sem_g = (sg0, sg1)
    sem_s = (ss0, ss1)

    _zero_fill(fbuf.at[0], K)
    for t in range(RPT // K):
        pltpu.sync_copy(fbuf.at[0], acc.at[pl.ds(sid * RPT + t * K, K)])

    @pl.when(cid == 1)
    def _preset():
        one0 = jnp.where(jnp.arange(16, dtype=jnp.int32) == 0,
                         jnp.float32(1.0), jnp.float32(0.0))
        _zero_fill(fbuf.at[1], K)

        def body(i, carry):
            fbuf[0, i, pl.ds(D_EDGE, 16)] = one0
            fbuf[1, i, pl.ds(D_EDGE, 16)] = one0
            return carry

        lax.fori_loop(0, K, body, 0)

    plsc.subcore_barrier()

    @pl.when(cid == 0)
    def _edges_a():
        def unpack_half(h, fb, base):
            gb = gbuf.at[h]

            def body(i, c):
                for j in range(XW // 16):
                    w = gb[i, pl.ds(j * 16, 16)]
                    v = plsc.bitcast(w, jnp.bfloat16)
                    lo, hi = plsc.unpack(
                        v, format=plsc.PackFormat.INTERLEAVED,
                        preferred_element_type=jnp.float32)
                    fb[base + i, pl.ds(j * 32, 16)] = lo
                    fb[base + i, pl.ds(j * 32 + 16, 16)] = hi
                return c

            lax.fori_loop(0, KH, body, 0)

        def glaunch(b, h):
            return pltpu.async_copy(
                xp_h.at[sidxg.at[pl.ds(b * K + h * KH, KH)]],
                gbuf.at[h], sem_g[h])

        def group(gg, carry):
            geb = pl.multiple_of(sid * EPT + gg * GK, GK)
            grow = pl.multiple_of(sid * NCH + gg * G, 8)
            pltpu.sync_copy(src_h.at[pl.ds(geb, GK)], sidxg)
            pltpu.sync_copy(dst2_h.at[pl.ds(grow, G)], didxg)
            cp_g = [glaunch(0, 0), glaunch(0, 1)]
            cp_s = [None, None]
            for b in range(G):
                cur = b % 2
                if b >= 2:
                    cp_s[cur].wait()
                fb = fbuf.at[cur]
                cp_g[0].wait()
                unpack_half(0, fb, 0)
                if b + 1 < G:
                    cp_g[0] = glaunch(b + 1, 0)
                cp_g[1].wait()
                unpack_half(1, fb, KH)
                if b + 1 < G:
                    cp_g[1] = glaunch(b + 1, 1)
                cp_s[cur] = pltpu.async_copy(
                    fb, acc.at[didxg.at[b]], sem_s[cur], add=True)
            cp_s[0].wait()
            cp_s[1].wait()
            return carry

        lax.fori_loop(0, NGRP, group, 0)

    @pl.when(cid == 1)
    def _edges_b():
        def group(gg, carry):
            grow = pl.multiple_of(sid * NCH + gg * G, 8)
            pltpu.sync_copy(dst2_h.at[pl.ds(grow, G)], didxg)
            cp_s = [None, None]
            for b in range(G):
                cur = b % 2
                prow = pl.multiple_of(
                    jnp.minimum((sid * EPT + (gg * G + b) * K) // 8,
                                N_EDGES // 8 - K // 8), K // 8)
                pltpu.sync_copy(efp_h.at[pl.ds(prow, K // 8)], packed)
                if b >= 2:
                    cp_s[cur].wait()
                buf = fbuf.at[cur]

                def repack(i, c):
                    buf[i, pl.ds(0, 16)] = packed[i // 8, pl.ds((i % 8) * 16, 16)]
                    return c

                lax.fori_loop(0, K, repack, 0)
                cp_s[cur] = pltpu.async_copy(
                    buf, acc.at[didxg.at[b]], sem_s[cur], add=True)
            cp_s[0].wait()
            cp_s[1].wait()
            return carry

        lax.fori_loop(0, NGRP, group, 0)

    plsc.subcore_barrier()

    base = sid * RPT

    @pl.when(cid == 0)
    def _drain_a():
        pltpu.sync_copy(acc.at[pl.ds(base, RPT)], outa_h.at[pl.ds(base, RPT)])

    @pl.when(cid == 1)
    def _drain_b():
        pltpu.sync_copy(acc.at[pl.ds(base, RPT)], outb_h.at[pl.ds(base, RPT)])


_sc_call = pl.kernel(
    _sc_segsum,
    out_type=(jax.ShapeDtypeStruct((NROW, DA), jnp.float32),
              jax.ShapeDtypeStruct((NROW, DA), jnp.float32)),
    mesh=plsc.VectorSubcoreMesh(core_axis_name="c", subcore_axis_name="s",
                                num_cores=NC, num_subcores=NS),
    compiler_params=pltpu.CompilerParams(use_tc_tiling_on_sc=False,
                                         needs_layout_passes=False),
    scratch_types=[
        pltpu.VMEM_SHARED((NROW, DA), jnp.float32),
        pltpu.VMEM((GK,), jnp.int32),
        pltpu.VMEM((G, K), jnp.int32),
        pltpu.VMEM((2, KH, XW), jnp.int32),
        pltpu.VMEM((2, K, DA), jnp.float32),
        pltpu.VMEM((K // 8, DA), jnp.float32),
        pltpu.SemaphoreType.DMA,
        pltpu.SemaphoreType.DMA,
        pltpu.SemaphoreType.DMA,
        pltpu.SemaphoreType.DMA,
    ],
)


def _tc_finish(a_ref, bb_ref, w_ref, bias_ref, o_ref):
    a = a_ref[...]
    bb = bb_ref[...]
    cnt = bb[:, D_EDGE:D_EDGE + 1]
    h = jnp.dot(a, w_ref[:D_FEAT, :], preferred_element_type=jnp.float32)
    h = h + jnp.dot(bb[:, :D_EDGE], w_ref[D_FEAT:, :],
                    preferred_element_type=jnp.float32)
    h = h + cnt * bias_ref[...]
    o_ref[...] = h / jnp.maximum(cnt, 1.0)


_tc_call = pl.pallas_call(
    _tc_finish,
    grid=(N_NODES // BM,),
    in_specs=[
        pl.BlockSpec((BM, DA), lambda i: (i, 0)),
        pl.BlockSpec((BM, DA), lambda i: (i, 0)),
        pl.BlockSpec((D_FEAT + D_EDGE, D_OUT), lambda i: (0, 0)),
        pl.BlockSpec((1, D_OUT), lambda i: (0, 0)),
    ],
    out_specs=pl.BlockSpec((BM, D_OUT), lambda i: (i, 0)),
    out_shape=jax.ShapeDtypeStruct((N_NODES, D_OUT), jnp.float32),
)


def kernel(x, edge_index, edge_features, W, b):
    npad = EPAD - N_EDGES
    src = jnp.concatenate([edge_index[0].astype(jnp.int32),
                           jnp.zeros((npad,), jnp.int32)])
    dst = jnp.concatenate([edge_index[1].astype(jnp.int32),
                           jnp.full((npad,), TRASH, jnp.int32)])
    dst2 = dst.reshape(EPAD // K, K)
    xb = x.astype(jnp.bfloat16).reshape(N_NODES, 4, 2, 16)
    xp = jax.lax.bitcast_convert_type(xb.transpose(0, 1, 3, 2),
                                      jnp.int32).reshape(N_NODES, XW)
    efp = edge_features.astype(jnp.float32).reshape(N_EDGES // 8, 128)
    pa, pb = _sc_call(src, dst2, xp, efp)
    return _tc_call(pa, pb, W, b.reshape(1, D_OUT))
